# Initial kernel scaffold; baseline (speedup 1.0000x reference)
import jax
import jax.numpy as jnp
from jax.experimental import pallas as pl


def kernel(hidden_states, Wr, e_score_correction_bias, W1, W3, W2):
    raise NotImplementedError("write your pallas kernel here")


# speedup vs baseline: 1.5331x; 1.5331x over previous
import functools

import jax
import jax.numpy as jnp
from jax.experimental import pallas as pl

T = 2048
D = 1024
FF = 2048
E = 8
Z = 2
NE = E + Z
EPAD = 128
K = 2
NFF = 4
FFB = FF // NFF


def _router_body(x_ref, wr_ref, bias_ref, comb_ref):
    x = x_ref[...]
    wr = wr_ref[...]
    logits = jax.lax.dot_general(
        x, wr, (((1,), (0,)), ((), ())),
        preferred_element_type=jnp.float32,
        precision=jax.lax.Precision.DEFAULT,
    )
    lane = jax.lax.broadcasted_iota(jnp.int32, (T, EPAD), 1)
    valid = lane < NE
    neg = jnp.float32(-1e30)
    lm = jnp.where(valid, logits, neg)
    m = jnp.max(lm, axis=1, keepdims=True)
    ex = jnp.where(valid, jnp.exp(lm - m), 0.0)
    scores = ex / jnp.sum(ex, axis=1, keepdims=True)
    sel = jnp.where(valid, scores + bias_ref[...], neg)
    m1 = jnp.max(sel, axis=1, keepdims=True)
    i1 = jnp.min(jnp.where(sel == m1, lane, EPAD), axis=1, keepdims=True)
    sel2 = jnp.where(lane == i1, neg, sel)
    m2 = jnp.max(sel2, axis=1, keepdims=True)
    i2 = jnp.min(jnp.where(sel2 == m2, lane, EPAD), axis=1, keepdims=True)
    picked = (lane == i1) | (lane == i2)
    comb_ref[...] = jnp.where(picked, scores, 0.0)


def _expert_body(x_ref, w1_ref, w3_ref, w2_ref, comb_ref, out_ref):
    e = pl.program_id(0)
    f = pl.program_id(1)
    lane = jax.lax.broadcasted_iota(jnp.int32, (T, EPAD), 1)
    comb = comb_ref[...]

    @pl.when((e == 0) & (f == 0))
    def _init():
        zw = jnp.sum(jnp.where((lane >= E) & (lane < NE), comb, 0.0),
                     axis=1, keepdims=True)
        out_ref[...] = zw * x_ref[...]

    xb = x_ref[...].astype(jnp.bfloat16)
    w1 = w1_ref[0].astype(jnp.bfloat16)
    w3 = w3_ref[0].astype(jnp.bfloat16)
    w2 = w2_ref[0].astype(jnp.bfloat16)
    dn = (((1,), (0,)), ((), ()))
    a = jax.lax.dot_general(xb, w1, dn, preferred_element_type=jnp.float32)
    b = jax.lax.dot_general(xb, w3, dn, preferred_element_type=jnp.float32)
    h = (a * (1.0 / (1.0 + jnp.exp(-a))) * b).astype(jnp.bfloat16)
    y = jax.lax.dot_general(h, w2, dn, preferred_element_type=jnp.float32)
    w_col = jnp.sum(jnp.where(lane == e, comb, 0.0), axis=1, keepdims=True)
    out_ref[...] += w_col * y


@jax.jit
def kernel(hidden_states, Wr, e_score_correction_bias, W1, W3, W2):
    wr_pad = jnp.zeros((D, EPAD), jnp.float32).at[:, :NE].set(Wr)
    bias_pad = jnp.zeros((1, EPAD), jnp.float32).at[0, :NE].set(
        e_score_correction_bias)

    comb = pl.pallas_call(
        _router_body,
        out_shape=jax.ShapeDtypeStruct((T, EPAD), jnp.float32),
    )(hidden_states, wr_pad, bias_pad)

    out = pl.pallas_call(
        _expert_body,
        grid=(E, NFF),
        in_specs=[
            pl.BlockSpec((T, D), lambda e, f: (0, 0)),
            pl.BlockSpec((1, D, FFB), lambda e, f: (e, 0, f)),
            pl.BlockSpec((1, D, FFB), lambda e, f: (e, 0, f)),
            pl.BlockSpec((1, FFB, D), lambda e, f: (e, f, 0)),
            pl.BlockSpec((T, EPAD), lambda e, f: (0, 0)),
        ],
        out_specs=pl.BlockSpec((T, D), lambda e, f: (0, 0)),
        out_shape=jax.ShapeDtypeStruct((T, D), jnp.float32),
    )(hidden_states, W1, W3, W2, comb)
    return out
